# Initial kernel scaffold; baseline (speedup 1.0000x reference)
import jax
import jax.numpy as jnp
from jax.experimental import pallas as pl


def kernel(means3d, opacity, scale3d, features, viewmatrix, projmatrix):
    raise NotImplementedError("write your pallas kernel here")


# speedup vs baseline: 2.8068x; 2.8068x over previous
import functools
import math

import jax
import jax.numpy as jnp
from jax.experimental import pallas as pl
from jax.experimental.pallas import tpu as pltpu

_N = 4096
_H = 256
_W = 256
_TILE = 64
_FOVX = math.radians(60.0)
_FOVY = math.radians(60.0)
_FOCAL_X = _W / (2.0 * math.tan(_FOVX * 0.5))
_FOCAL_Y = _H / (2.0 * math.tan(_FOVY * 0.5))
_RADII_MULT = 5.0
_BKGD = 1.0
_G = 128
_NCHUNK = _N // _G
_P = _TILE * _TILE
_HI = jax.lax.Precision.HIGHEST


def _bf(x):
    return x.astype(jnp.bfloat16).astype(jnp.float32)


def _project_kernel(mt_ref, sc_ref, op_ref, view_ref, viewb_ref, projb_ref,
                    rows_ref, radii_ref):
    mb0 = _bf(mt_ref[0:1, :])
    mb1 = _bf(mt_ref[1:2, :])
    mb2 = _bf(mt_ref[2:3, :])
    vb = [[viewb_ref[j, k] for k in range(4)] for j in range(4)]
    pb = [[projb_ref[j, k] for k in range(4)] for j in range(4)]
    pv = [mb0 * vb[0][k] + mb1 * vb[1][k] + mb2 * vb[2][k] + vb[3][k]
          for k in range(4)]
    pvb = [_bf(x) for x in pv]
    ph = [pvb[0] * pb[0][k] + pvb[1] * pb[1][k] + pvb[2] * pb[2][k]
          + pvb[3] * pb[3][k] for k in range(4)]
    pw = 1.0 / (ph[3] + 1e-6)
    mx = ((ph[0] * pw + 1.0) * _W - 1.0) * 0.5
    my = ((ph[1] * pw + 1.0) * _H - 1.0) * 0.5
    depth = pv[2]
    tz = mb0 * vb[0][2] + mb1 * vb[1][2] + mb2 * vb[2][2] + view_ref[3, 2]
    sx = sc_ref[0:1, :] * _FOCAL_X / tz
    sy = sc_ref[0:1, :] * _FOCAL_Y / tz
    smax = jnp.maximum(sx, sy)
    s2 = smax * smax
    radii = smax * _RADII_MULT
    rminx = jnp.clip(mx - radii, 0.0, _W - 1.0)
    rminy = jnp.clip(my - radii, 0.0, _H - 1.0)
    rmaxx = jnp.clip(mx + radii, 0.0, _W - 1.0)
    rmaxy = jnp.clip(my + radii, 0.0, _H - 1.0)
    rows_ref[:, :] = jnp.concatenate(
        [mx, my, s2, op_ref[0:1, :], rminx, rminy, rmaxx, rmaxy, depth],
        axis=0)
    radii_ref[:, :] = radii


def _split3(x):
    a = x.astype(jnp.bfloat16).astype(jnp.float32)
    r = x - a
    b = r.astype(jnp.bfloat16).astype(jnp.float32)
    c = r - b
    return a, b, c


def _sort_kernel(rows_ref, dcol_ref, feat_ref, srows_ref, smat_ref):
    d_row = rows_ref[8:9, :]
    d_col = dcol_ref[:, :]
    i_row = jax.lax.broadcasted_iota(jnp.int32, (1, _N), 1)
    i_col = jax.lax.broadcasted_iota(jnp.int32, (_N, 1), 0)
    cb = 512
    rc = jnp.zeros((_N, 1), jnp.float32)
    rr = jnp.zeros((1, _N), jnp.float32)
    for c in range(_N // cb):
        dj = d_row[:, c * cb:(c + 1) * cb]
        ij = i_row[:, c * cb:(c + 1) * cb]
        before = (dj < d_col) | ((dj == d_col) & (ij < i_col))
        rc = rc + jnp.sum(before.astype(jnp.float32), axis=1, keepdims=True)
        di = d_col[c * cb:(c + 1) * cb, :]
        ii = i_col[c * cb:(c + 1) * cb, :]
        before2 = (di < d_row) | ((di == d_row) & (ii < i_row))
        rr = rr + jnp.sum(before2.astype(jnp.float32), axis=0, keepdims=True)
    c8 = rows_ref[0:8, :]
    ones = jnp.ones((_N, 1), jnp.float32)
    zeros = jnp.zeros((_N, 3), jnp.float32)
    cm = jnp.concatenate([feat_ref[:, :], d_col, ones, zeros], axis=1)
    c8p = _split3(c8)
    cmp_ = _split3(cm)
    kr = jax.lax.broadcasted_iota(jnp.int32, (1, _G), 1)
    kc = jax.lax.broadcasted_iota(jnp.int32, (_G, 1), 0)

    def body(kb, _):
        base = kb * _G
        krow = (kr + base).astype(jnp.float32)
        kcol = (kc + base).astype(jnp.float32)
        et = (rc == krow).astype(jnp.float32)
        e = (rr == kcol).astype(jnp.float32)
        srows_ref[:, pl.ds(base, _G)] = sum(
            jnp.dot(part, et, preferred_element_type=jnp.float32,
                    precision=_HI) for part in c8p)
        smat_ref[pl.ds(base, _G), :] = sum(
            jnp.dot(e, part, preferred_element_type=jnp.float32,
                    precision=_HI) for part in cmp_)
        return 0

    jax.lax.fori_loop(0, _N // _G, body, 0)


def _render_kernel(rows_ref, mat_ref, out_ref):
    tid = pl.program_id(0)
    u = ((tid % 4) * _TILE).astype(jnp.float32)
    v = ((tid // 4) * _TILE).astype(jnp.float32)
    lin = jax.lax.broadcasted_iota(jnp.int32, (_P, 1), 0)
    xcol = u + (lin % _TILE).astype(jnp.float32)
    ycol = v + (lin // _TILE).astype(jnp.float32)

    def body(c, carry):
        t_run, acc = carry
        blk = rows_ref[:, pl.ds(c * _G, _G)]
        mx = blk[0:1, :]
        my = blk[1:2, :]
        s2 = blk[2:3, :]
        op = blk[3:4, :]
        tlx = jnp.maximum(blk[4:5, :], u)
        tly = jnp.maximum(blk[5:6, :], v)
        brx = jnp.minimum(blk[6:7, :], u + (_TILE - 1.0))
        bry = jnp.minimum(blk[7:8, :], v + (_TILE - 1.0))
        mask = (brx > tlx) & (bry > tly)
        opeff = jnp.where(mask, op, 0.0)
        neg_inv = -0.5 / s2
        dx = xcol - mx
        dy = ycol - my
        d2 = dx * dx + dy * dy
        gw = jnp.exp(d2 * neg_inv)
        a = jnp.minimum(gw * opeff, 0.99)
        om = 1.0 - a
        cp = om
        for k in (1, 2, 4, 8, 16, 32, 64):
            cp = cp * jnp.concatenate(
                [jnp.ones((_P, k), jnp.float32), cp[:, :_G - k]], axis=1)
        excl = jnp.concatenate(
            [jnp.ones((_P, 1), jnp.float32), cp[:, :_G - 1]], axis=1)
        w = (t_run * excl) * a
        mc = mat_ref[pl.ds(c * _G, _G), :]
        acc = acc + jnp.dot(w, mc, preferred_element_type=jnp.float32,
                            precision=_HI)
        t_run = t_run * cp[:, _G - 1:_G]
        return t_run, acc

    t0 = jnp.ones((_P, 1), jnp.float32)
    acc0 = jnp.zeros((_P, 8), jnp.float32)
    _, acc = jax.lax.fori_loop(0, _NCHUNK, body, (t0, acc0))
    acca = acc[:, 4:5]
    rgb = jnp.clip(acc[:, 0:3] + (1.0 - acca) * _BKGD, 0.0, 1.0)
    out_ref[0] = jnp.concatenate(
        [rgb, acc[:, 3:4], acca, jnp.zeros((_P, 3), jnp.float32)], axis=1)


@functools.partial(jax.jit, static_argnames=())
def kernel(means3d, opacity, scale3d, features, viewmatrix, projmatrix):
    mt = means3d.T
    sc_row = scale3d.reshape(1, _N)
    op_row = opacity.reshape(1, _N)
    view_b = viewmatrix.astype(jnp.bfloat16).astype(jnp.float32)
    proj_b = projmatrix.astype(jnp.bfloat16).astype(jnp.float32)
    rows, radii_row = pl.pallas_call(
        _project_kernel,
        out_shape=[jax.ShapeDtypeStruct((9, _N), jnp.float32),
                   jax.ShapeDtypeStruct((1, _N), jnp.float32)],
        in_specs=[pl.BlockSpec(memory_space=pltpu.VMEM),
                  pl.BlockSpec(memory_space=pltpu.VMEM),
                  pl.BlockSpec(memory_space=pltpu.VMEM),
                  pl.BlockSpec(memory_space=pltpu.SMEM),
                  pl.BlockSpec(memory_space=pltpu.SMEM),
                  pl.BlockSpec(memory_space=pltpu.SMEM)],
    )(mt, sc_row, op_row, viewmatrix, view_b, proj_b)
    d_col = rows[8].reshape(_N, 1)
    srows, smat = pl.pallas_call(
        _sort_kernel,
        out_shape=[jax.ShapeDtypeStruct((8, _N), jnp.float32),
                   jax.ShapeDtypeStruct((_N, 8), jnp.float32)],
    )(rows, d_col, features)
    out = pl.pallas_call(
        _render_kernel,
        grid=(16,),
        in_specs=[pl.BlockSpec((8, _N), lambda i: (0, 0)),
                  pl.BlockSpec((_N, 8), lambda i: (0, 0))],
        out_specs=pl.BlockSpec((1, _P, 8), lambda i: (i, 0, 0)),
        out_shape=jax.ShapeDtypeStruct((16, _P, 8), jnp.float32),
    )(srows, smat)
    img = out.reshape(4, 4, _TILE, _TILE, 8).transpose(0, 2, 1, 3, 4)
    img = img.reshape(_H, _W, 8)
    render_color = img[:, :, 0:3]
    render_depth = img[:, :, 3:4]
    render_alpha = img[:, :, 4:5]
    radii = radii_row.reshape(_N)
    return render_color, render_depth, render_alpha, radii
